# baseline (speedup 1.0000x reference)
import functools

import jax
import jax.numpy as jnp
from jax import lax
from jax.experimental import pallas as pl
from jax.experimental.pallas import tpu as pltpu


def _body(adj_hbm, x_ref, w_ref, b_ref, out_ref, buf_ref, sem_ref, *,
          block_m, nbuf):
    i = pl.program_id(0)
    nsteps = pl.num_programs(0)

    def start(slab, slot):
        pltpu.make_async_copy(
            adj_hbm.at[pl.ds(slab * block_m, block_m), :],
            buf_ref.at[slot],
            sem_ref.at[slot],
        ).start()

    @pl.when(i == 0)
    def _prologue():
        for s in range(nbuf - 1):
            start(s, s)

    nxt = i + nbuf - 1

    @pl.when(nxt < nsteps)
    def _issue():
        start(nxt, lax.rem(nxt, nbuf))

    slot = lax.rem(i, nbuf)
    pltpu.make_async_copy(
        adj_hbm.at[pl.ds(i * block_m, block_m), :],
        buf_ref.at[slot],
        sem_ref.at[slot],
    ).wait()

    h = jnp.dot(buf_ref[slot], x_ref[...], preferred_element_type=jnp.float32)
    y = lax.dot_general(
        h, w_ref[...], (((1,), (1,)), ((), ())),
        preferred_element_type=jnp.float32,
    )
    out_ref[...] = jnp.maximum(y + b_ref[...], 0.0)


@functools.partial(jax.jit, static_argnames=("block_m", "nbuf", "interpret"))
def _fused_graph_layer(x, adj, W, b2d, block_m, nbuf, interpret=False):
    n, d_in = x.shape
    d_out = W.shape[0]
    grid = (n // block_m,)
    return pl.pallas_call(
        functools.partial(_body, block_m=block_m, nbuf=nbuf),
        grid=grid,
        in_specs=[
            pl.BlockSpec(memory_space=pl.ANY),
            pl.BlockSpec((n, d_in), lambda i: (0, 0)),
            pl.BlockSpec((d_out, d_in), lambda i: (0, 0)),
            pl.BlockSpec((1, d_out), lambda i: (0, 0)),
        ],
        out_specs=pl.BlockSpec((block_m, d_out), lambda i: (i, 0)),
        out_shape=jax.ShapeDtypeStruct((n, d_out), jnp.float32),
        scratch_shapes=[
            pltpu.VMEM((nbuf, block_m, n), jnp.float32),
            pltpu.SemaphoreType.DMA((nbuf,)),
        ],
        interpret=interpret,
    )(adj, x, W, b2d)


def kernel(x, adj, W, b):
    b2d = b.reshape(1, -1)
    return _fused_graph_layer(x, adj, W, b2d, block_m=200, nbuf=6)


if __name__ == "__main__":
    import numpy as np
    key = jax.random.key(0)
    k1, k2, k3, k4 = jax.random.split(key, 4)
    n, d = 1024, 128
    x = jax.random.normal(k1, (n, d), dtype=jnp.float32)
    adj = jax.random.uniform(k2, (n, n), dtype=jnp.float32)
    W = jax.random.normal(k3, (d, d), dtype=jnp.float32) * 0.1
    b = jax.random.normal(k4, (d,), dtype=jnp.float32)
    got = _fused_graph_layer(x, adj, W, b.reshape(1, -1), block_m=128, nbuf=4,
                             interpret=True)
    want = jax.nn.relu((adj @ x) @ W.T + b)
    err = float(jnp.max(jnp.abs(got - want)))
    print("max_abs_err:", err)
    assert err < 1e-3
    print("interpret-mode check PASSED")


# speedup vs baseline: 1.0108x; 1.0108x over previous
import functools

import jax
import jax.numpy as jnp
from jax import lax
from jax.experimental import pallas as pl
from jax.experimental.pallas import tpu as pltpu


def _body(adj_hbm, x_ref, w_ref, b_ref, out_ref, buf_ref, sem_ref, *,
          block_m, nbuf):
    i = pl.program_id(0)
    nsteps = pl.num_programs(0)

    def start(slab, slot):
        pltpu.make_async_copy(
            adj_hbm.at[pl.ds(slab * block_m, block_m), :],
            buf_ref.at[slot],
            sem_ref.at[slot],
        ).start()

    @pl.when(i == 0)
    def _prologue():
        for s in range(nbuf - 1):
            start(s, s)

    nxt = i + nbuf - 1

    @pl.when(nxt < nsteps)
    def _issue():
        start(nxt, lax.rem(nxt, nbuf))

    slot = lax.rem(i, nbuf)
    pltpu.make_async_copy(
        adj_hbm.at[pl.ds(i * block_m, block_m), :],
        buf_ref.at[slot],
        sem_ref.at[slot],
    ).wait()

    h = jnp.dot(buf_ref[slot], x_ref[...], preferred_element_type=jnp.float32)
    y = lax.dot_general(
        h, w_ref[...], (((1,), (1,)), ((), ())),
        preferred_element_type=jnp.float32,
    )
    out_ref[...] = jnp.maximum(y + b_ref[...], 0.0)


@functools.partial(jax.jit, static_argnames=("block_m", "nbuf", "interpret"))
def _fused_graph_layer(x, adj, W, b2d, block_m, nbuf, interpret=False):
    n, d_in = x.shape
    d_out = W.shape[0]
    grid = (n // block_m,)
    return pl.pallas_call(
        functools.partial(_body, block_m=block_m, nbuf=nbuf),
        grid=grid,
        in_specs=[
            pl.BlockSpec(memory_space=pl.ANY),
            pl.BlockSpec((n, d_in), lambda i: (0, 0)),
            pl.BlockSpec((d_out, d_in), lambda i: (0, 0)),
            pl.BlockSpec((1, d_out), lambda i: (0, 0)),
        ],
        out_specs=pl.BlockSpec((block_m, d_out), lambda i: (i, 0)),
        out_shape=jax.ShapeDtypeStruct((n, d_out), jnp.float32),
        scratch_shapes=[
            pltpu.VMEM((nbuf, block_m, n), jnp.float32),
            pltpu.SemaphoreType.DMA((nbuf,)),
        ],
        interpret=interpret,
    )(adj, x, W, b2d)


def kernel(x, adj, W, b):
    b2d = b.reshape(1, -1)
    return _fused_graph_layer(x, adj, W, b2d, block_m=400, nbuf=3)


if __name__ == "__main__":
    import numpy as np
    key = jax.random.key(0)
    k1, k2, k3, k4 = jax.random.split(key, 4)
    n, d = 1024, 128
    x = jax.random.normal(k1, (n, d), dtype=jnp.float32)
    adj = jax.random.uniform(k2, (n, n), dtype=jnp.float32)
    W = jax.random.normal(k3, (d, d), dtype=jnp.float32) * 0.1
    b = jax.random.normal(k4, (d,), dtype=jnp.float32)
    got = _fused_graph_layer(x, adj, W, b.reshape(1, -1), block_m=128, nbuf=4,
                             interpret=True)
    want = jax.nn.relu((adj @ x) @ W.T + b)
    err = float(jnp.max(jnp.abs(got - want)))
    print("max_abs_err:", err)
    assert err < 1e-3
    print("interpret-mode check PASSED")


# speedup vs baseline: 1.0434x; 1.0322x over previous
import functools

import jax
import jax.numpy as jnp
from jax import lax
from jax.experimental import pallas as pl
from jax.experimental.pallas import tpu as pltpu


def _fused_body(adj_ref, x_ref, w_ref, b_ref, out_ref):
    h = jnp.dot(adj_ref[...], x_ref[...], preferred_element_type=jnp.float32)
    y = lax.dot_general(
        h, w_ref[...], (((1,), (1,)), ((), ())),
        preferred_element_type=jnp.float32,
    )
    out_ref[...] = jnp.maximum(y + b_ref[...], 0.0)


@functools.partial(jax.jit, static_argnames=("block_m",))
def _fused_graph_layer(x, adj, W, b2d, block_m):
    n, d_in = x.shape
    d_out = W.shape[0]
    grid = (pl.cdiv(n, block_m),)
    return pl.pallas_call(
        _fused_body,
        grid=grid,
        in_specs=[
            pl.BlockSpec((block_m, n), lambda i: (i, 0)),
            pl.BlockSpec((n, d_in), lambda i: (0, 0)),
            pl.BlockSpec((d_out, d_in), lambda i: (0, 0)),
            pl.BlockSpec((1, d_out), lambda i: (0, 0)),
        ],
        out_specs=pl.BlockSpec((block_m, d_out), lambda i: (i, 0)),
        out_shape=jax.ShapeDtypeStruct((n, d_out), jnp.float32),
    )(adj, x, W, b2d)


def kernel(x, adj, W, b):
    b2d = b.reshape(1, -1)
    return _fused_graph_layer(x, adj, W, b2d, block_m=400)


# speedup vs baseline: 1.0440x; 1.0006x over previous
import functools

import jax
import jax.numpy as jnp
from jax import lax
from jax.experimental import pallas as pl
from jax.experimental.pallas import tpu as pltpu


def _fused_body(adj_ref, x_ref, w_ref, b_ref, out_ref):
    h = jnp.dot(adj_ref[...], x_ref[...], preferred_element_type=jnp.float32)
    y = lax.dot_general(
        h, w_ref[...], (((1,), (1,)), ((), ())),
        preferred_element_type=jnp.float32,
    )
    out_ref[...] = jnp.maximum(y + b_ref[...], 0.0)


@functools.partial(jax.jit, static_argnames=("block_m",))
def _fused_graph_layer(x, adj, W, b2d, block_m):
    n, d_in = x.shape
    d_out = W.shape[0]
    grid = (pl.cdiv(n, block_m),)
    return pl.pallas_call(
        _fused_body,
        grid=grid,
        in_specs=[
            pl.BlockSpec((block_m, n), lambda i: (i, 0)),
            pl.BlockSpec((n, d_in), lambda i: (0, 0)),
            pl.BlockSpec((d_out, d_in), lambda i: (0, 0)),
            pl.BlockSpec((1, d_out), lambda i: (0, 0)),
        ],
        out_specs=pl.BlockSpec((block_m, d_out), lambda i: (i, 0)),
        out_shape=jax.ShapeDtypeStruct((n, d_out), jnp.float32),
        compiler_params=pltpu.CompilerParams(
            dimension_semantics=("parallel",),
        ),
    )(adj, x, W, b2d)


def kernel(x, adj, W, b):
    b2d = b.reshape(1, -1)
    return _fused_graph_layer(x, adj, W, b2d, block_m=400)
